# baseline (speedup 1.0000x reference)
import functools

import jax
import jax.numpy as jnp
from jax import lax
from jax.experimental import pallas as pl
from jax.experimental.pallas import tpu as pltpu
from jax.experimental.pallas import tpu_sc as plsc

_B, _T, _ENC, _FILT, _K, _MEL = 16, 512, 256, 256, 3, 4096
_NC, _NS = 2, 16
_NW = _NC * _NS
_WPB = 4
_MROWS = _MEL // _WPB
_CH = 64
_NCHUNK = _MROWS // _CH
_LANES = 16


def _dp_body(x_ref, w1_ref, b1_ref, g1_ref, be1_ref, w2_ref, b2_ref, g2_ref,
             be2_ref, lw_ref, lb_ref, out_ref):
    x = x_ref[0]

    def conv(h, w_ref, b_ref):
        z = jnp.zeros((1, h.shape[1]), jnp.float32)
        hp = jnp.concatenate([z, h[:-1, :]], axis=0)
        hn = jnp.concatenate([h[1:, :], z], axis=0)
        y = (jnp.dot(hp, w_ref[0], preferred_element_type=jnp.float32)
             + jnp.dot(h, w_ref[1], preferred_element_type=jnp.float32)
             + jnp.dot(hn, w_ref[2], preferred_element_type=jnp.float32))
        return y + b_ref[...]

    def ln(h, g_ref, b_ref):
        mu = jnp.mean(h, axis=1, keepdims=True)
        d = h - mu
        var = jnp.mean(d * d, axis=1, keepdims=True)
        return d * lax.rsqrt(var + 1e-5) * g_ref[...] + b_ref[...]

    h = ln(jnp.maximum(conv(x, w1_ref, b1_ref), 0.0), g1_ref, be1_ref)
    h = ln(jnp.maximum(conv(h, w2_ref, b2_ref), 0.0), g2_ref, be2_ref)
    out_ref[0, 0, :] = jnp.sum(h * lw_ref[...], axis=1) + lb_ref[0, 0]


def _duration_predictor(x, w1t, b1, g1, be1, w2t, b2, g2, be2, lw, lb):
    full3 = pl.BlockSpec((_K, _ENC, _FILT), lambda b: (0, 0, 0))
    vec = pl.BlockSpec((1, _FILT), lambda b: (0, 0))
    out3 = pl.pallas_call(
        _dp_body,
        grid=(_B,),
        in_specs=[
            pl.BlockSpec((1, _T, _ENC), lambda b: (b, 0, 0)),
            full3, vec, vec, vec,
            full3, vec, vec, vec,
            vec, pl.BlockSpec((1, 1), lambda b: (0, 0)),
        ],
        out_specs=pl.BlockSpec((1, 1, _T), lambda b: (b, 0, 0)),
        out_shape=jax.ShapeDtypeStruct((_B, 1, _T), jnp.float32),
    )(x, w1t, b1, g1, be1, w2t, b2, g2, be2, lw, lb)
    return out3.reshape(_B, _T)


def _length_regulator(xz, dur):
    mesh = plsc.VectorSubcoreMesh(core_axis_name="c", subcore_axis_name="s",
                                  num_cores=_NC, num_subcores=_NS)

    @functools.partial(
        pl.kernel,
        out_type=jax.ShapeDtypeStruct((_B * _MEL, _ENC), jnp.float32),
        mesh=mesh,
        scratch_types=[
            pltpu.VMEM((_T,), jnp.int32),
            pltpu.VMEM((_T,), jnp.int32),
            pltpu.VMEM((_CH,), jnp.int32),
            pltpu.VMEM((_CH,), jnp.int32),
            pltpu.VMEM((2, _CH, _ENC), jnp.float32),
            pltpu.VMEM((_CH, _ENC), jnp.float32),
            pltpu.VMEM_SHARED(((_NS // _WPB) * _T + 8, _ENC),
                              jnp.float32),
            pltpu.SemaphoreType.DMA,
            pltpu.SemaphoreType.DMA,
            pltpu.SemaphoreType.DMA,
            pltpu.SemaphoreType.DMA,
        ],
        compiler_params=pltpu.CompilerParams(needs_layout_passes=False),
    )
    def lr(xz_hbm, dur_hbm, out_hbm, dur_v, cum_v, idxa, idxb, bufs, zbuf,
           stage, sem0, sem1, semo0, semo1):
        cid = lax.axis_index("c")
        sid = lax.axis_index("s")
        wid = cid * _NS + sid
        b = wid // _WPB
        bb = b % (_NS // _WPB)
        half = wid % _WPB
        m0 = half * _MROWS

        pltpu.sync_copy(dur_hbm.at[b], dur_v)

        _ZROW = (_NS // _WPB) * _T
        pltpu.sync_copy(
            xz_hbm.at[pl.ds(b * _T + half * (_T // _WPB), _T // _WPB)],
            stage.at[pl.ds(bb * _T + half * (_T // _WPB), _T // _WPB)])

        @pl.when(sid == 0)
        def _write_zero_row():
            for k in range(_ENC // _LANES):
                bufs[0, 0, pl.ds(k * _LANES, _LANES)] = jnp.zeros(
                    (_LANES,), jnp.float32)
            pltpu.sync_copy(bufs.at[0].at[pl.ds(0, 1)],
                            stage.at[pl.ds(_ZROW, 1)])

        plsc.subcore_barrier()

        carry = jnp.int32(0)
        for i in range(_T // _LANES):
            v = dur_v[pl.ds(i * _LANES, _LANES)]
            cum_v[pl.ds(i * _LANES, _LANES)] = plsc.cumsum(v) + carry
            carry = carry + jnp.sum(v)
        total = plsc.load_gather(
            cum_v, [jnp.full((_LANES,), _T - 1, jnp.int32)])[0]

        def zero_body(r, acc):
            pltpu.async_copy(stage.at[pl.ds(jnp.int32(_ZROW), 1)],
                             zbuf.at[pl.ds(r, 1)], sem0)
            return acc

        lax.fori_loop(0, _CH, zero_body, 0)
        pltpu.make_async_copy(xz_hbm.at[pl.ds(0, _CH)], zbuf, sem0).wait()

        zero_row = jnp.int32(_ZROW)
        row_base = bb * _T

        def fill_idx(idx_ref, c):
            cz = (m0 + c * _CH) >= total

            @pl.when(cz)
            def _fill_zero():
                def zbody(k, acc):
                    idx_ref[pl.ds(k * _LANES, _LANES)] = (
                        jnp.zeros((_LANES,), jnp.int32) + zero_row)
                    return acc

                lax.fori_loop(0, _CH // _LANES, zbody, 0)

            @pl.when(jnp.logical_not(cz))
            def _fill_search():
                def body(k, acc):
                    m = m0 + c * _CH + k * _LANES + lax.iota(jnp.int32, _LANES)
                    lo = jnp.zeros((_LANES,), jnp.int32)
                    hi = jnp.full((_LANES,), _T, jnp.int32)
                    for _ in range(10):
                        mid = jnp.minimum(jnp.right_shift(lo + hi, 1), _T - 1)
                        val = plsc.load_gather(cum_v, [mid])
                        pred = val <= m
                        lo = jnp.where(pred, mid + 1, lo)
                        hi = jnp.where(pred, hi, mid)
                    idx_ref[pl.ds(k * _LANES, _LANES)] = jnp.where(
                        lo >= _T, zero_row, row_base + lo)
                    return acc

                lax.fori_loop(0, _CH // _LANES, body, 0)

        out_base = b * _MEL + m0
        idxs = (idxa, idxb)
        sems = (sem0, sem1)

        def fire_rows(idx_ref, buf, sem):
            def body(k, acc):
                v = idx_ref[pl.ds(k * _LANES, _LANES)]
                for j in range(_LANES):
                    pltpu.async_copy(stage.at[pl.ds(v[j], 1)],
                                     buf.at[pl.ds(k * _LANES + j, 1)], sem)
                return acc

            lax.fori_loop(0, _CH // _LANES, body, 0)

        def drain(buf, sem):
            pltpu.make_async_copy(xz_hbm.at[pl.ds(0, _CH)], buf, sem).wait()

        semos = (semo0, semo1)
        for c in range(_NCHUNK + 2):
            if c >= 2:
                p = c - 2
                pz = (m0 + p * _CH) >= total

                @pl.when(jnp.logical_not(pz))
                def _send_gathered(p=p):
                    drain(bufs.at[p % 2], sems[p % 2])
                    pltpu.async_copy(
                        bufs.at[p % 2],
                        out_hbm.at[pl.ds(out_base + p * _CH, _CH)],
                        semos[p % 2])

                @pl.when(pz)
                def _send_zeros(p=p):
                    pltpu.async_copy(
                        zbuf, out_hbm.at[pl.ds(out_base + p * _CH, _CH)],
                        semos[p % 2])

            if c < _NCHUNK:
                cz = (m0 + c * _CH) >= total
                if c >= 2:
                    pltpu.make_async_copy(
                        bufs.at[c % 2],
                        out_hbm.at[pl.ds(out_base + (c - 2) * _CH, _CH)],
                        semos[c % 2]).wait()

                @pl.when(jnp.logical_not(cz))
                def _gather_chunk(c=c):
                    fill_idx(idxs[c % 2], c)
                    fire_rows(idxs[c % 2], bufs.at[c % 2], sems[c % 2])
        for p in (_NCHUNK - 2, _NCHUNK - 1):
            pltpu.make_async_copy(
                bufs.at[p % 2],
                out_hbm.at[pl.ds(out_base + p * _CH, _CH)],
                semos[p % 2]).wait()

    return lr(xz, dur)


def kernel(x, conv1_w, conv1_b, ln1_g, ln1_b, conv2_w, conv2_b, ln2_g, ln2_b,
           lin_w, lin_b, length_target, mel_max_length):
    del mel_max_length
    out2 = _length_regulator(x.reshape(_B * _T, _ENC), length_target)

    w1t = jnp.transpose(conv1_w, (2, 1, 0))
    w2t = jnp.transpose(conv2_w, (2, 1, 0))
    dpo = _duration_predictor(
        x, w1t, conv1_b.reshape(1, _FILT), ln1_g.reshape(1, _FILT),
        ln1_b.reshape(1, _FILT), w2t, conv2_b.reshape(1, _FILT),
        ln2_g.reshape(1, _FILT), ln2_b.reshape(1, _FILT),
        lin_w.reshape(1, _FILT), lin_b.reshape(1, 1))

    return (out2.reshape(_B, _MEL, _ENC), dpo)


# speedup vs baseline: 1.0966x; 1.0966x over previous
import functools

import jax
import jax.numpy as jnp
from jax import lax
from jax.experimental import pallas as pl
from jax.experimental.pallas import tpu as pltpu
from jax.experimental.pallas import tpu_sc as plsc

_B, _T, _ENC, _FILT, _K, _MEL = 16, 512, 256, 256, 3, 4096
_NC, _NS = 2, 16
_NW = _NC * _NS
_WPB = 4
_MROWS = _MEL // _WPB
_CH = 128
_NCHUNK = _MROWS // _CH
_LANES = 16


def _dp_body(x_ref, w1_ref, b1_ref, g1_ref, be1_ref, w2_ref, b2_ref, g2_ref,
             be2_ref, lw_ref, lb_ref, out_ref):
    x = x_ref[0]

    def conv(h, w_ref, b_ref):
        z = jnp.zeros((1, h.shape[1]), jnp.float32)
        hp = jnp.concatenate([z, h[:-1, :]], axis=0)
        hn = jnp.concatenate([h[1:, :], z], axis=0)
        y = (jnp.dot(hp, w_ref[0], preferred_element_type=jnp.float32)
             + jnp.dot(h, w_ref[1], preferred_element_type=jnp.float32)
             + jnp.dot(hn, w_ref[2], preferred_element_type=jnp.float32))
        return y + b_ref[...]

    def ln(h, g_ref, b_ref):
        mu = jnp.mean(h, axis=1, keepdims=True)
        d = h - mu
        var = jnp.mean(d * d, axis=1, keepdims=True)
        return d * lax.rsqrt(var + 1e-5) * g_ref[...] + b_ref[...]

    h = ln(jnp.maximum(conv(x, w1_ref, b1_ref), 0.0), g1_ref, be1_ref)
    h = ln(jnp.maximum(conv(h, w2_ref, b2_ref), 0.0), g2_ref, be2_ref)
    out_ref[0, 0, :] = jnp.sum(h * lw_ref[...], axis=1) + lb_ref[0, 0]


def _duration_predictor(x, w1t, b1, g1, be1, w2t, b2, g2, be2, lw, lb):
    full3 = pl.BlockSpec((_K, _ENC, _FILT), lambda b: (0, 0, 0))
    vec = pl.BlockSpec((1, _FILT), lambda b: (0, 0))
    out3 = pl.pallas_call(
        _dp_body,
        grid=(_B,),
        in_specs=[
            pl.BlockSpec((1, _T, _ENC), lambda b: (b, 0, 0)),
            full3, vec, vec, vec,
            full3, vec, vec, vec,
            vec, pl.BlockSpec((1, 1), lambda b: (0, 0)),
        ],
        out_specs=pl.BlockSpec((1, 1, _T), lambda b: (b, 0, 0)),
        out_shape=jax.ShapeDtypeStruct((_B, 1, _T), jnp.float32),
    )(x, w1t, b1, g1, be1, w2t, b2, g2, be2, lw, lb)
    return out3.reshape(_B, _T)


def _length_regulator(xz, dur):
    mesh = plsc.VectorSubcoreMesh(core_axis_name="c", subcore_axis_name="s",
                                  num_cores=_NC, num_subcores=_NS)

    @functools.partial(
        pl.kernel,
        out_type=jax.ShapeDtypeStruct((_B * _MEL, _ENC), jnp.float32),
        mesh=mesh,
        scratch_types=[
            pltpu.VMEM((_T,), jnp.int32),
            pltpu.VMEM((_T,), jnp.int32),
            pltpu.VMEM((_CH,), jnp.int32),
            pltpu.VMEM((_CH,), jnp.int32),
            pltpu.VMEM((2, _CH, _ENC), jnp.float32),
            pltpu.VMEM_SHARED(((_NS // _WPB) * _T + 8, _ENC),
                              jnp.float32),
            pltpu.SemaphoreType.DMA,
            pltpu.SemaphoreType.DMA,
            pltpu.SemaphoreType.DMA,
            pltpu.SemaphoreType.DMA,
        ],
        compiler_params=pltpu.CompilerParams(needs_layout_passes=False),
    )
    def lr(xz_hbm, dur_hbm, out_hbm, dur_v, cum_v, idxa, idxb, bufs,
           stage, sem0, sem1, semo0, semo1):
        cid = lax.axis_index("c")
        sid = lax.axis_index("s")
        wid = cid * _NS + sid
        b = wid // _WPB
        bb = b % (_NS // _WPB)
        half = wid % _WPB
        m0 = half * _MROWS

        pltpu.sync_copy(dur_hbm.at[b], dur_v)

        _ZROW = (_NS // _WPB) * _T
        pltpu.sync_copy(
            xz_hbm.at[pl.ds(b * _T + half * (_T // _WPB), _T // _WPB)],
            stage.at[pl.ds(bb * _T + half * (_T // _WPB), _T // _WPB)])

        @pl.when(sid == 0)
        def _write_zero_row():
            for k in range(_ENC // _LANES):
                bufs[0, 0, pl.ds(k * _LANES, _LANES)] = jnp.zeros(
                    (_LANES,), jnp.float32)
            pltpu.sync_copy(bufs.at[0].at[pl.ds(0, 1)],
                            stage.at[pl.ds(_ZROW, 1)])

        plsc.subcore_barrier()

        carry = jnp.int32(0)
        for i in range(_T // _LANES):
            v = dur_v[pl.ds(i * _LANES, _LANES)]
            cum_v[pl.ds(i * _LANES, _LANES)] = plsc.cumsum(v) + carry
            carry = carry + jnp.sum(v)
        total = plsc.load_gather(
            cum_v, [jnp.full((_LANES,), _T - 1, jnp.int32)])[0]

        zero_row = jnp.int32(_ZROW)
        row_base = bb * _T

        def fill_idx(idx_ref, c):
            cz = (m0 + c * _CH) >= total

            @pl.when(cz)
            def _fill_zero():
                def zbody(k, acc):
                    idx_ref[pl.ds(k * _LANES, _LANES)] = (
                        jnp.zeros((_LANES,), jnp.int32) + zero_row)
                    return acc

                lax.fori_loop(0, _CH // _LANES, zbody, 0)

            @pl.when(jnp.logical_not(cz))
            def _fill_search():
                def body(k, acc):
                    m = m0 + c * _CH + k * _LANES + lax.iota(jnp.int32, _LANES)
                    lo = jnp.zeros((_LANES,), jnp.int32)
                    hi = jnp.full((_LANES,), _T, jnp.int32)
                    for _ in range(10):
                        mid = jnp.minimum(jnp.right_shift(lo + hi, 1), _T - 1)
                        val = plsc.load_gather(cum_v, [mid])
                        pred = val <= m
                        lo = jnp.where(pred, mid + 1, lo)
                        hi = jnp.where(pred, hi, mid)
                    idx_ref[pl.ds(k * _LANES, _LANES)] = jnp.where(
                        lo >= _T, zero_row, row_base + lo)
                    return acc

                lax.fori_loop(0, _CH // _LANES, body, 0)

        out_base = b * _MEL + m0
        idxs = (idxa, idxb)
        sems = (sem0, sem1)

        def fire_rows(idx_ref, buf, sem):
            def body(k, acc):
                v = idx_ref[pl.ds(k * _LANES, _LANES)]
                for j in range(_LANES):
                    pltpu.async_copy(stage.at[pl.ds(v[j], 1)],
                                     buf.at[pl.ds(k * _LANES + j, 1)], sem)
                return acc

            lax.fori_loop(0, _CH // _LANES, body, 0)

        def drain(buf, sem):
            pltpu.make_async_copy(xz_hbm.at[pl.ds(0, _CH)], buf, sem).wait()

        semos = (semo0, semo1)
        outcps = [None, None]
        for c in range(_NCHUNK + 2):
            if c >= 2:
                p = c - 2
                drain(bufs.at[p % 2], sems[p % 2])
                outcps[p % 2] = pltpu.async_copy(
                    bufs.at[p % 2],
                    out_hbm.at[pl.ds(out_base + p * _CH, _CH)],
                    semos[p % 2])
            if c < _NCHUNK:
                fill_idx(idxs[c % 2], c)
                if c >= 2:
                    outcps[c % 2].wait()
                fire_rows(idxs[c % 2], bufs.at[c % 2], sems[c % 2])
        outcps[(_NCHUNK - 2) % 2].wait()
        outcps[(_NCHUNK - 1) % 2].wait()

    return lr(xz, dur)


def kernel(x, conv1_w, conv1_b, ln1_g, ln1_b, conv2_w, conv2_b, ln2_g, ln2_b,
           lin_w, lin_b, length_target, mel_max_length):
    del mel_max_length
    out2 = _length_regulator(x.reshape(_B * _T, _ENC), length_target)

    w1t = jnp.transpose(conv1_w, (2, 1, 0))
    w2t = jnp.transpose(conv2_w, (2, 1, 0))
    dpo = _duration_predictor(
        x, w1t, conv1_b.reshape(1, _FILT), ln1_g.reshape(1, _FILT),
        ln1_b.reshape(1, _FILT), w2t, conv2_b.reshape(1, _FILT),
        ln2_g.reshape(1, _FILT), ln2_b.reshape(1, _FILT),
        lin_w.reshape(1, _FILT), lin_b.reshape(1, 1))

    return (out2.reshape(_B, _MEL, _ENC), dpo)
